# baseline (speedup 1.0000x reference)
import functools

import jax
import jax.numpy as jnp
from jax import lax
from jax.experimental import pallas as pl
from jax.experimental.pallas import tpu as pltpu
from jax.experimental.pallas import tpu_sc as plsc

BATCH = 16384
NUM_FIELDS = 26
FIELD_SIZE = 100000

NC = 2
NS = 16
NW = NC * NS
B_PER_W = BATCH // NW
N_IDX = B_PER_W * NUM_FIELDS


def _body(x_ref, table_ref, out_ref, x_v, idx_v, rows_v, out_v, sem):
    wid = lax.axis_index("s") * NC + lax.axis_index("c")
    base = wid * N_IDX

    pltpu.sync_copy(x_ref.at[pl.ds(base, N_IDX)], x_v)

    lanes26 = lax.iota(jnp.int32, 16) * NUM_FIELDS

    def build(t, _):
        f = t // (B_PER_W // 16)
        c2 = t % (B_PER_W // 16)
        xpos = lanes26 + (c2 * 16 * NUM_FIELDS + f)
        xv = plsc.load_gather(x_v, [xpos])
        idx_v[pl.ds(t * 16, 16)] = xv + f * FIELD_SIZE
        return 0

    lax.fori_loop(0, NUM_FIELDS * (B_PER_W // 16), build, 0, unroll=4)

    pltpu.async_copy(table_ref.at[idx_v], rows_v, sem).wait()

    def reduce_chunk(c2, _):
        def add_f(f, acc):
            q = f * B_PER_W + c2 * 16
            return acc + rows_v[pl.ds(q, 16)]

        acc = lax.fori_loop(
            0, NUM_FIELDS, add_f, jnp.zeros((16,), jnp.float32), unroll=4
        )
        out_v[pl.ds(c2 * 16, 16)] = acc
        return 0

    lax.fori_loop(0, B_PER_W // 16, reduce_chunk, 0)

    pltpu.sync_copy(out_v, out_ref.at[pl.ds(wid * B_PER_W, B_PER_W)])


@jax.jit
def kernel(x, table, bias):
    mesh = plsc.VectorSubcoreMesh(core_axis_name="c", subcore_axis_name="s")
    k = pl.kernel(
        _body,
        out_type=jax.ShapeDtypeStruct((BATCH,), jnp.float32),
        mesh=mesh,
        compiler_params=pltpu.CompilerParams(
            needs_layout_passes=False, use_tc_tiling_on_sc=False
        ),
        scratch_types=[
            pltpu.VMEM((N_IDX,), jnp.int32),
            pltpu.VMEM((N_IDX,), jnp.int32),
            pltpu.VMEM((N_IDX,), jnp.float32),
            pltpu.VMEM((B_PER_W,), jnp.float32),
            pltpu.SemaphoreType.DMA,
        ],
    )
    out = k(x.reshape(-1), lax.reshape(table, (table.shape[0],), dimensions=(1, 0)))
    return out.reshape(BATCH, 1) + bias[None, :]


# speedup vs baseline: 1.0019x; 1.0019x over previous
import functools

import jax
import jax.numpy as jnp
from jax import lax
from jax.experimental import pallas as pl
from jax.experimental.pallas import tpu as pltpu
from jax.experimental.pallas import tpu_sc as plsc

BATCH = 16384
NUM_FIELDS = 26
FIELD_SIZE = 100000

NC = 2
NS = 16
NW = NC * NS
B_PER_W = BATCH // NW
N_IDX = B_PER_W * NUM_FIELDS


def _body(x_ref, table_ref, out_ref, x_v, idx_v, rows_v, out_v, sem):
    wid = lax.axis_index("s") * NC + lax.axis_index("c")
    base = wid * N_IDX

    pltpu.sync_copy(x_ref.at[pl.ds(base, N_IDX)], x_v)

    lanes26 = lax.iota(jnp.int32, 16) * NUM_FIELDS

    def build(t, _):
        f = t // (B_PER_W // 16)
        c2 = t % (B_PER_W // 16)
        xpos = lanes26 + (c2 * 16 * NUM_FIELDS + f)
        xv = plsc.load_gather(x_v, [xpos])
        idx_v[pl.ds(t * 16, 16)] = xv + f * FIELD_SIZE
        return 0

    lax.fori_loop(0, NUM_FIELDS * (B_PER_W // 16), build, 0, unroll=4)

    pltpu.async_copy(table_ref.at[idx_v], rows_v, sem).wait()

    def reduce_chunk(c2, _):
        def add_f(f, acc):
            q = f * B_PER_W + c2 * 16
            return acc + rows_v[pl.ds(q, 16)]

        acc = lax.fori_loop(
            0, NUM_FIELDS, add_f, jnp.zeros((16,), jnp.float32), unroll=4
        )
        out_v[pl.ds(c2 * 16, 16)] = acc
        return 0

    lax.fori_loop(0, B_PER_W // 16, reduce_chunk, 0)

    pltpu.sync_copy(out_v, out_ref.at[pl.ds(wid * B_PER_W, B_PER_W)])


@jax.jit
def kernel(x, table, bias):
    mesh = plsc.VectorSubcoreMesh(core_axis_name="c", subcore_axis_name="s")
    k = pl.kernel(
        _body,
        out_type=jax.ShapeDtypeStruct((BATCH,), jnp.float32),
        mesh=mesh,
        compiler_params=pltpu.CompilerParams(
            needs_layout_passes=False, use_tc_tiling_on_sc=False
        ),
        scratch_types=[
            pltpu.VMEM((N_IDX,), jnp.int32),
            pltpu.VMEM((N_IDX,), jnp.int32),
            pltpu.VMEM((N_IDX,), jnp.float32),
            pltpu.VMEM((B_PER_W,), jnp.float32),
            pltpu.SemaphoreType.DMA,
        ],
    )
    flat = jnp.einsum("vi,i->v", table, jnp.ones((1,), jnp.float32))
    out = k(x.reshape(-1), flat)
    return out.reshape(BATCH, 1) + bias[None, :]
